# baseline (speedup 1.0000x reference)
import jax
import jax.numpy as jnp
from jax import lax
from jax.experimental import pallas as pl
from jax.experimental.pallas import tpu as pltpu

_EPS = 1e-6


def _body(mu_ref, sg_ref, y_ref, pit_ref, sel_ade_ref, sel_fde_ref,
          reg_ref, cls_ref):
    K, T2, Bb = mu_ref.shape
    T = T2 // 2
    yt = y_ref[...]
    l2 = None
    dfde = None
    for t in range(T):
        dx = mu_ref[:, 2 * t, :] - yt[2 * t][None]
        dy = mu_ref[:, 2 * t + 1, :] - yt[2 * t + 1][None]
        dist = jnp.sqrt(dx * dx + dy * dy)
        l2 = dist if l2 is None else l2 + dist
        if t == T - 1:
            dfde = dist

    kio = lax.broadcasted_iota(jnp.int32, (K, Bb), 0)
    minv = jnp.min(l2, axis=0)
    best = jnp.min(jnp.where(l2 == minv[None], kio, K), axis=0)
    mask = (kio == best[None]).astype(jnp.float32)
    minf = jnp.min(dfde, axis=0)
    bestf = jnp.min(jnp.where(dfde == minf[None], kio, K), axis=0)
    maskf = (kio == bestf[None]).astype(jnp.float32)

    sel_mu = mask[0][None] * mu_ref[0]
    sel_sg = mask[0][None] * sg_ref[0]
    sel_f = maskf[0][None] * mu_ref[0]
    for k in range(1, K):
        sel_mu = sel_mu + mask[k][None] * mu_ref[k]
        sel_sg = sel_sg + mask[k][None] * sg_ref[k]
        sel_f = sel_f + maskf[k][None] * mu_ref[k]
    sel_ade_ref[...] = sel_mu
    sel_fde_ref[...] = sel_f

    sc = jnp.maximum(sel_sg, _EPS)
    nll = jnp.log(2.0 * sc) + jnp.abs(yt - sel_mu) / sc
    reg_part = jnp.sum(nll)

    z = l2 * (-1.0 / T)
    zm = jnp.max(z, axis=0)
    ez = jnp.exp(z - zm[None])
    st = ez / jnp.sum(ez, axis=0)[None]
    pit = pit_ref[...]
    pm = jnp.max(pit, axis=0)
    lse = jnp.log(jnp.sum(jnp.exp(pit - pm[None]), axis=0)) + pm
    ce = jnp.sum(st * (lse[None] - pit), axis=0)
    cls_part = jnp.sum(ce)

    @pl.when(pl.program_id(0) == 0)
    def _init():
        reg_ref[...] = jnp.zeros_like(reg_ref)
        cls_ref[...] = jnp.zeros_like(cls_ref)

    reg_ref[...] = reg_ref[...] + jnp.reshape(reg_part, (1, 1))
    cls_ref[...] = cls_ref[...] + jnp.reshape(cls_part, (1, 1))


def _run(mu_t, sg_t, y_t, pit, K, B, T2, Bb, interpret=False):
    return pl.pallas_call(
        _body,
        grid=(B // Bb,),
        in_specs=[
            pl.BlockSpec((K, T2, Bb), lambda i: (0, 0, i)),
            pl.BlockSpec((K, T2, Bb), lambda i: (0, 0, i)),
            pl.BlockSpec((T2, Bb), lambda i: (0, i)),
            pl.BlockSpec((K, Bb), lambda i: (0, i)),
        ],
        out_specs=[
            pl.BlockSpec((T2, Bb), lambda i: (0, i)),
            pl.BlockSpec((T2, Bb), lambda i: (0, i)),
            pl.BlockSpec((1, 1), lambda i: (0, 0)),
            pl.BlockSpec((1, 1), lambda i: (0, 0)),
        ],
        out_shape=[
            jax.ShapeDtypeStruct((T2, B), jnp.float32),
            jax.ShapeDtypeStruct((T2, B), jnp.float32),
            jax.ShapeDtypeStruct((1, 1), jnp.float32),
            jax.ShapeDtypeStruct((1, 1), jnp.float32),
        ],
        interpret=interpret,
    )(mu_t, sg_t, y_t, pit)


def kernel(out_mu, out_sigma, out_pi, y, pre_obs):
    K, B, T, _ = out_mu.shape
    T2 = 2 * T
    mu_t = jnp.transpose(out_mu.reshape(K, B, T2), (0, 2, 1))
    sg_t = jnp.transpose(out_sigma.reshape(K, B, T2), (0, 2, 1))

    y_t = jnp.transpose(y, (0, 2, 1)).reshape(T2, B)
    pit = jnp.transpose(out_pi, (1, 0))
    Bb = 512 if B % 512 == 0 else B
    sel_ade, sel_fde, reg, cls = _run(mu_t, sg_t, y_t, pit, K, B, T2, Bb)
    loss = reg[0, 0] / (B * T2) + cls[0, 0] / B
    sk = jnp.transpose(sel_ade.reshape(T, 2, B), (0, 2, 1))
    skf = jnp.transpose(sel_fde.reshape(T, 2, B), (0, 2, 1))
    tra_ade = jnp.concatenate([pre_obs, sk], axis=0)
    tra_fde = jnp.concatenate([pre_obs, skf], axis=0)
    return (loss, tra_ade, tra_fde)


# speedup vs baseline: 1.0668x; 1.0668x over previous
import jax
import jax.numpy as jnp
from jax import lax
from jax.experimental import pallas as pl
from jax.experimental.pallas import tpu as pltpu

_EPS = 1e-6


def _body(mu_ref, sg_ref, y_ref, pit_ref, sel_ade_ref, sel_fde_ref,
          reg_ref, cls_ref):
    K, T2, Bb = mu_ref.shape
    T = T2 // 2
    yt = y_ref[...]
    l2 = None
    dfde = None
    for t in range(T):
        dx = mu_ref[:, 2 * t, :] - yt[2 * t][None]
        dy = mu_ref[:, 2 * t + 1, :] - yt[2 * t + 1][None]
        dist = jnp.sqrt(dx * dx + dy * dy)
        l2 = dist if l2 is None else l2 + dist
        if t == T - 1:
            dfde = dist

    kio = lax.broadcasted_iota(jnp.int32, (K, Bb), 0)
    minv = jnp.min(l2, axis=0)
    best = jnp.min(jnp.where(l2 == minv[None], kio, K), axis=0)
    mask = (kio == best[None]).astype(jnp.float32)
    minf = jnp.min(dfde, axis=0)
    bestf = jnp.min(jnp.where(dfde == minf[None], kio, K), axis=0)
    maskf = (kio == bestf[None]).astype(jnp.float32)

    sel_mu = mask[0][None] * mu_ref[0]
    sel_sg = mask[0][None] * sg_ref[0]
    sel_f = maskf[0][None] * mu_ref[0]
    for k in range(1, K):
        sel_mu = sel_mu + mask[k][None] * mu_ref[k]
        sel_sg = sel_sg + mask[k][None] * sg_ref[k]
        sel_f = sel_f + maskf[k][None] * mu_ref[k]
    sel_ade_ref[...] = sel_mu
    sel_fde_ref[...] = sel_f

    sc = jnp.maximum(sel_sg, _EPS)
    nll = jnp.log(2.0 * sc) + jnp.abs(yt - sel_mu) / sc
    reg_part = jnp.sum(nll)

    z = l2 * (-1.0 / T)
    zm = jnp.max(z, axis=0)
    ez = jnp.exp(z - zm[None])
    st = ez / jnp.sum(ez, axis=0)[None]
    pit = pit_ref[...]
    pm = jnp.max(pit, axis=0)
    lse = jnp.log(jnp.sum(jnp.exp(pit - pm[None]), axis=0)) + pm
    ce = jnp.sum(st * (lse[None] - pit), axis=0)
    cls_part = jnp.sum(ce)

    @pl.when(pl.program_id(0) == 0)
    def _init():
        reg_ref[...] = jnp.zeros_like(reg_ref)
        cls_ref[...] = jnp.zeros_like(cls_ref)

    reg_ref[...] = reg_ref[...] + jnp.reshape(reg_part, (1, 1))
    cls_ref[...] = cls_ref[...] + jnp.reshape(cls_part, (1, 1))


def _run(mu_t, sg_t, y_t, pit, K, B, T2, Bb, interpret=False):
    return pl.pallas_call(
        _body,
        grid=(B // Bb,),
        in_specs=[
            pl.BlockSpec((K, T2, Bb), lambda i: (0, 0, i)),
            pl.BlockSpec((K, T2, Bb), lambda i: (0, 0, i)),
            pl.BlockSpec((T2, Bb), lambda i: (0, i)),
            pl.BlockSpec((K, Bb), lambda i: (0, i)),
        ],
        out_specs=[
            pl.BlockSpec((T2, Bb), lambda i: (0, i)),
            pl.BlockSpec((T2, Bb), lambda i: (0, i)),
            pl.BlockSpec((1, 1), lambda i: (0, 0)),
            pl.BlockSpec((1, 1), lambda i: (0, 0)),
        ],
        out_shape=[
            jax.ShapeDtypeStruct((T2, B), jnp.float32),
            jax.ShapeDtypeStruct((T2, B), jnp.float32),
            jax.ShapeDtypeStruct((1, 1), jnp.float32),
            jax.ShapeDtypeStruct((1, 1), jnp.float32),
        ],
        interpret=interpret,
    )(mu_t, sg_t, y_t, pit)


def kernel(out_mu, out_sigma, out_pi, y, pre_obs):
    K, B, T, _ = out_mu.shape
    T2 = 2 * T
    mu_t = jnp.transpose(out_mu.reshape(K, B, T2), (0, 2, 1))
    sg_t = jnp.transpose(out_sigma.reshape(K, B, T2), (0, 2, 1))

    y_t = jnp.transpose(y, (0, 2, 1)).reshape(T2, B)
    pit = jnp.transpose(out_pi, (1, 0))
    Bb = 1024 if B % 1024 == 0 else B
    sel_ade, sel_fde, reg, cls = _run(mu_t, sg_t, y_t, pit, K, B, T2, Bb)
    loss = reg[0, 0] / (B * T2) + cls[0, 0] / B
    sk = jnp.transpose(sel_ade.reshape(T, 2, B), (0, 2, 1))
    skf = jnp.transpose(sel_fde.reshape(T, 2, B), (0, 2, 1))
    tra_ade = jnp.concatenate([pre_obs, sk], axis=0)
    tra_fde = jnp.concatenate([pre_obs, skf], axis=0)
    return (loss, tra_ade, tra_fde)
